# baseline (speedup 1.0000x reference)
import jax
import jax.numpy as jnp
from jax import lax
from jax.experimental import pallas as pl
from jax.experimental.pallas import tpu as pltpu
from jax.experimental.pallas import tpu_sc as plsc

_F32 = jnp.float32

_NUM_CORES = 2
_NUM_SUBCORES = 16
_NW = _NUM_CORES * _NUM_SUBCORES
_B = 128


def _mesh():
    return plsc.VectorSubcoreMesh(
        core_axis_name="c", subcore_axis_name="s",
        num_cores=_NUM_CORES, num_subcores=_NUM_SUBCORES)


def _sc_degree(ep2, zeros16, ones_b16, n, np_, e):
    per_tile = e // _NW
    nb = per_tile // _B
    rpt = np_ // _NUM_SUBCORES

    def body(ep_hbm, ones_hbm, zeros_hbm, out_hbm,
             acc, ia, ib, ones, sa, sb):
        cid = lax.axis_index("c")
        sid = lax.axis_index("s")
        r0 = sid * rpt
        pltpu.sync_copy(zeros_hbm, acc.at[pl.ds(r0, rpt)])
        pltpu.sync_copy(ones_hbm, ones)
        plsc.subcore_barrier()
        jb = (cid * _NUM_SUBCORES + sid) * nb

        def istart(k, buf, sem):
            pltpu.async_copy(ep_hbm.at[jb + k], buf, sem)

        def iwait(buf, sem):
            pltpu.make_async_copy(ep_hbm.at[jb], buf, sem).wait()

        def scat(buf):
            pltpu.sync_copy(ones, acc.at[buf.at[0]], add=True)

        istart(0, ia, sa)

        def pair(g, carry):
            k = 2 * g
            iwait(ia, sa)
            istart(k + 1, ib, sb)
            scat(ia)
            iwait(ib, sb)
            istart(k + 2, ia, sa)
            scat(ib)
            return carry

        lax.fori_loop(0, (nb - 1) // 2, pair, 0)
        iwait(ia, sa)
        scat(ia)
        plsc.subcore_barrier()
        pltpu.sync_copy(acc.at[pl.ds(r0, rpt)], out_hbm.at[cid, pl.ds(r0, rpt)])

    f = pl.kernel(
        body,
        out_type=jax.ShapeDtypeStruct((_NUM_CORES, np_, 16), _F32),
        mesh=_mesh(),
        compiler_params=pltpu.CompilerParams(use_tc_tiling_on_sc=False),
        scratch_types=[
            pltpu.VMEM_SHARED((np_, 16), _F32),
            pltpu.VMEM((2, _B), jnp.int32),
            pltpu.VMEM((2, _B), jnp.int32),
            pltpu.VMEM((_B, 16), _F32),
            pltpu.SemaphoreType.DMA,
            pltpu.SemaphoreType.DMA,
        ],
    )
    return f(ep2, ones_b16, zeros16)


def _stream_loop(t_hbm, ep_hbm, acc, ia, ib, b0, b1, sa, sb, sg, jb, nb):
    def istart(k, buf, sem):
        pltpu.async_copy(ep_hbm.at[jb + k], buf, sem)

    def iwait(buf, sem):
        pltpu.make_async_copy(ep_hbm.at[jb], buf, sem).wait()

    def gstart(ibuf, buf):
        pltpu.async_copy(t_hbm.at[ibuf.at[0]], buf, sg)

    def gwait(ibuf, buf):
        pltpu.make_async_copy(t_hbm.at[ibuf.at[0]], buf, sg).wait()

    def scat(ibuf, buf):
        pltpu.sync_copy(buf, acc.at[ibuf.at[1]], add=True)

    istart(0, ia, sa)
    istart(1, ib, sb)
    iwait(ia, sa)
    gstart(ia, b0)

    def pair(g, carry):
        k = 2 * g
        gwait(ia, b0)
        iwait(ib, sb)
        gstart(ib, b1)
        scat(ia, b0)
        istart(k + 2, ia, sa)
        gwait(ib, b1)
        iwait(ia, sa)
        gstart(ia, b0)
        scat(ib, b1)
        istart(k + 3, ib, sb)
        return carry

    n_pairs = (nb - 2) // 2 if nb % 2 == 0 else (nb - 1) // 2
    lax.fori_loop(0, n_pairs, pair, 0)

    if nb % 2 == 0:
        k = nb - 2
        gwait(ia, b0)
        iwait(ib, sb)
        gstart(ib, b1)
        scat(ia, b0)
        gwait(ib, b1)
        scat(ib, b1)
    else:
        gwait(ia, b0)
        scat(ia, b0)
        iwait(ib, sb)


def _sc_spmm_main(g, ep1, zeros_w, n, np_, e, w):
    nb = (e // _B) // _NUM_SUBCORES
    rpt = np_ // _NUM_SUBCORES

    def body(g_hbm, ep_hbm, zeros_hbm, out_hbm,
             acc, ia, ib, b0, b1, sa, sb, sg):
        cid = lax.axis_index("c")
        sid = lax.axis_index("s")
        r0 = sid * rpt
        pltpu.sync_copy(zeros_hbm, acc.at[pl.ds(r0, rpt)])
        plsc.subcore_barrier()
        jb = cid * (e // _B) + sid * nb
        _stream_loop(g_hbm, ep_hbm, acc, ia, ib, b0, b1, sa, sb, sg, jb, nb)
        plsc.subcore_barrier()
        pltpu.sync_copy(acc.at[pl.ds(r0, rpt)], out_hbm.at[cid, pl.ds(r0, rpt)])

    f = pl.kernel(
        body,
        out_type=jax.ShapeDtypeStruct((_NUM_CORES, np_, w), _F32),
        mesh=_mesh(),
        compiler_params=pltpu.CompilerParams(use_tc_tiling_on_sc=False),
        scratch_types=[
            pltpu.VMEM_SHARED((np_, w), _F32),
            pltpu.VMEM((2, _B), jnp.int32),
            pltpu.VMEM((2, _B), jnp.int32),
            pltpu.VMEM((_B, w), _F32),
            pltpu.VMEM((_B, w), _F32),
            pltpu.SemaphoreType.DMA,
            pltpu.SemaphoreType.DMA,
            pltpu.SemaphoreType.DMA,
        ],
    )
    return f(g, ep1, zeros_w)


def _sc_spmm_small(hw2, ep2, zeros16, n, np_, e, c_dim):
    nb = (e // _B) // _NW
    rpt = np_ // _NUM_SUBCORES

    def body(t_hbm, ep_hbm, zeros_hbm, out_hbm,
             acc, ia, ib, b0, b1, sa, sb, sg):
        cid = lax.axis_index("c")
        sid = lax.axis_index("s")
        r0 = sid * rpt
        pltpu.sync_copy(zeros_hbm, acc.at[pl.ds(r0, rpt)])
        plsc.subcore_barrier()
        jb = (cid * _NUM_SUBCORES + sid) * nb
        _stream_loop(t_hbm, ep_hbm, acc, ia, ib, b0, b1, sa, sb, sg, jb, nb)
        plsc.subcore_barrier()
        pltpu.sync_copy(acc.at[pl.ds(r0, rpt)], out_hbm.at[cid, pl.ds(r0, rpt)])

    f = pl.kernel(
        body,
        out_type=jax.ShapeDtypeStruct((_NUM_CORES, np_, c_dim), _F32),
        mesh=_mesh(),
        compiler_params=pltpu.CompilerParams(use_tc_tiling_on_sc=False),
        scratch_types=[
            pltpu.VMEM_SHARED((np_, c_dim), _F32),
            pltpu.VMEM((2, _B), jnp.int32),
            pltpu.VMEM((2, _B), jnp.int32),
            pltpu.VMEM((_B, c_dim), _F32),
            pltpu.VMEM((_B, c_dim), _F32),
            pltpu.SemaphoreType.DMA,
            pltpu.SemaphoreType.DMA,
            pltpu.SemaphoreType.DMA,
        ],
    )
    return f(hw2, ep2, zeros16)


def _tc_tables_body(x_ref, m_ref, deg_ref, g_ref, *, nblk):
    pid = pl.program_id(0)
    deg = deg_ref[0, :, 0:1] + deg_ref[1, :, 0:1]
    dis = jnp.where(deg > 0.0, lax.rsqrt(jnp.maximum(deg, 1e-30)), 0.0)
    xb = x_ref[...]
    x0 = jnp.where(jnp.isnan(xb), 0.0, xb)
    mb = m_ref[...]
    g1m = dis * mb
    g0 = g1m * x0
    bsz = g0.shape[0]
    lane16 = lax.broadcasted_iota(jnp.int32, (bsz, 16), 1)
    pad0 = jnp.zeros((bsz, 16), _F32)
    pad1 = jnp.where(lane16 == 0, jnp.broadcast_to(dis, (bsz, 16)), 0.0)
    blk0 = jnp.concatenate([g0, pad0], axis=1)
    blk1 = jnp.concatenate([g1m, pad1], axis=1)
    g_ref[...] = jnp.where(pid < nblk, blk0, blk1)


def _tc_tables(x, mask, deg16, n, d, w):
    bn = 1000
    nblk = n // bn
    import functools as _ft
    return pl.pallas_call(
        _ft.partial(_tc_tables_body, nblk=nblk),
        grid=(2 * nblk,),
        in_specs=[
            pl.BlockSpec((bn, d), lambda i: (lax.rem(i, nblk), 0)),
            pl.BlockSpec((bn, d), lambda i: (lax.rem(i, nblk), 0)),
            pl.BlockSpec((_NUM_CORES, bn, 16), lambda i: (0, lax.rem(i, nblk), 0)),
        ],
        out_specs=pl.BlockSpec((bn, w), lambda i: (i, 0)),
        out_shape=jax.ShapeDtypeStruct((2 * n, w), _F32),
    )(x, mask, deg16)


def _tc_mlp_body(p_ref, deg_ref, w1_ref, b1_ref, w2_ref, out_ref):
    deg = deg_ref[0, :, 0:1] + deg_ref[1, :, 0:1]
    dis = jnp.where(deg > 0.0, lax.rsqrt(jnp.maximum(deg, 1e-30)), 0.0)
    dis2 = lax.rsqrt(deg + 1.0)
    p0 = p_ref[0, :, 0:128]
    p1m = p_ref[1, :, 0:128]
    s1 = p_ref[1, :, 128:129]
    num = (dis * s1) * p0
    ratio = jnp.where(p1m != 0.0, num / jnp.where(p1m != 0.0, p1m, 1.0), 0.0)
    h = jnp.dot(ratio, w1_ref[...], preferred_element_type=_F32) + b1_ref[...]
    h = jnp.maximum(h, 0.0)
    hw = jnp.dot(h, w2_ref[...], preferred_element_type=_F32)
    out_ref[...] = dis2 * hw


def _tc_mlp(p, deg16, w1, b1, w2, n, np_, w, h_dim, c_dim):
    bn = 1000
    grid = (n // bn,)
    return pl.pallas_call(
        _tc_mlp_body,
        grid=grid,
        in_specs=[
            pl.BlockSpec((_NUM_CORES, bn, w), lambda i: (0, i, 0)),
            pl.BlockSpec((_NUM_CORES, bn, 16), lambda i: (0, i, 0)),
            pl.BlockSpec((128, h_dim), lambda i: (0, 0)),
            pl.BlockSpec((1, h_dim), lambda i: (0, 0)),
            pl.BlockSpec((h_dim, c_dim), lambda i: (0, 0)),
        ],
        out_specs=pl.BlockSpec((bn, c_dim), lambda i: (i, 0)),
        out_shape=jax.ShapeDtypeStruct((n, c_dim), _F32),
    )(p, deg16, w1, b1, w2)


def _tc_final_body(q_ref, hw2_ref, deg_ref, b2_ref, out_ref):
    deg = deg_ref[0, :, 0:1] + deg_ref[1, :, 0:1]
    dis2 = lax.rsqrt(deg + 1.0)
    y = dis2 * (q_ref[0] + q_ref[1] + hw2_ref[...]) + b2_ref[...]
    m = jnp.max(y, axis=1, keepdims=True)
    s = y - m
    out_ref[...] = s - jnp.log(jnp.sum(jnp.exp(s), axis=1, keepdims=True))


def _tc_final(q, hw2, deg16, b2, n, c_dim):
    bn = 1000
    grid = (n // bn,)
    return pl.pallas_call(
        _tc_final_body,
        grid=grid,
        in_specs=[
            pl.BlockSpec((_NUM_CORES, bn, c_dim), lambda i: (0, i, 0)),
            pl.BlockSpec((bn, c_dim), lambda i: (i, 0)),
            pl.BlockSpec((_NUM_CORES, bn, 16), lambda i: (0, i, 0)),
            pl.BlockSpec((1, c_dim), lambda i: (0, 0)),
        ],
        out_specs=pl.BlockSpec((bn, c_dim), lambda i: (i, 0)),
        out_shape=jax.ShapeDtypeStruct((n, c_dim), _F32),
    )(q, hw2, deg16, b2)


def kernel(x, edge_index, mask, W1, b1, W2, b2):
    n, d = x.shape
    e = edge_index.shape[1]
    h_dim = W1.shape[1]
    c_dim = W2.shape[1]
    w = d + 16
    np_ = ((n + _NUM_SUBCORES * 8 - 1) // (_NUM_SUBCORES * 8)) * (_NUM_SUBCORES * 8)
    rpt = np_ // _NUM_SUBCORES

    ep_ = ((e + _NW * _B - 1) // (_NW * _B)) * (_NW * _B)
    npad = ep_ - e
    row = edge_index[0]
    col = edge_index[1]
    nbt = ep_ // _B
    colg = jnp.reshape(jnp.concatenate([col, jnp.zeros((npad,), jnp.int32)]), (nbt, _B))
    rows = jnp.reshape(jnp.concatenate([row, jnp.full((npad,), n, jnp.int32)]), (nbt, _B))
    cols = jnp.reshape(jnp.concatenate([col, jnp.full((npad,), n, jnp.int32)]), (nbt, _B))
    pad4 = jnp.zeros((4, 2, _B), jnp.int32)
    ep0 = jnp.concatenate([jnp.stack([cols, rows], axis=1), pad4], axis=0)
    ep2 = jnp.concatenate([jnp.stack([colg, rows], axis=1), pad4], axis=0)
    ep1 = jnp.concatenate([
        jnp.stack([colg, rows], axis=1),
        jnp.stack([colg + n, rows], axis=1),
        pad4,
    ], axis=0)

    zeros16 = jnp.zeros((rpt, 16), _F32)
    zeros_w = jnp.zeros((rpt, w), _F32)
    ones_b16 = jnp.ones((_B, 16), _F32)

    deg16 = _sc_degree(ep0, zeros16, ones_b16, n, np_, ep_)
    g = _tc_tables(x, mask, deg16, n, d, w)
    p = _sc_spmm_main(g, ep1, zeros_w, n, np_, ep_, w)
    hw2 = _tc_mlp(p, deg16, W1, jnp.reshape(b1, (1, h_dim)), W2, n, np_, w, h_dim, c_dim)
    q = _sc_spmm_small(hw2, ep2, zeros16, n, np_, ep_, c_dim)
    out = _tc_final(q, hw2, deg16, jnp.reshape(b2, (1, c_dim)), n, c_dim)
    return out


# speedup vs baseline: 1.4159x; 1.4159x over previous
import jax
import jax.numpy as jnp
from jax import lax
from jax.experimental import pallas as pl
from jax.experimental.pallas import tpu as pltpu
from jax.experimental.pallas import tpu_sc as plsc

_F32 = jnp.float32

_NUM_CORES = 2
_NUM_SUBCORES = 16
_NW = _NUM_CORES * _NUM_SUBCORES
_B = 128


def _mesh():
    return plsc.VectorSubcoreMesh(
        core_axis_name="c", subcore_axis_name="s",
        num_cores=_NUM_CORES, num_subcores=_NUM_SUBCORES)


def _sc_degree(ep2, zeros16, ones_b16, n, np_, e):
    per_tile = e // _NW
    nb = per_tile // _B
    rpt = np_ // _NUM_SUBCORES

    def body(ep_hbm, ones_hbm, zeros_hbm, out_hbm,
             acc, ia, ib, ones, sa, sb):
        cid = lax.axis_index("c")
        sid = lax.axis_index("s")
        r0 = sid * rpt
        pltpu.sync_copy(zeros_hbm, acc.at[pl.ds(r0, rpt)])
        pltpu.sync_copy(ones_hbm, ones)
        plsc.subcore_barrier()
        jb = (cid * _NUM_SUBCORES + sid) * nb

        def istart(k, buf, sem):
            pltpu.async_copy(ep_hbm.at[jb + k], buf, sem)

        def iwait(buf, sem):
            pltpu.make_async_copy(ep_hbm.at[jb], buf, sem).wait()

        def scat(buf):
            pltpu.sync_copy(ones, acc.at[buf.at[0]], add=True)

        istart(0, ia, sa)

        def pair(g, carry):
            k = 2 * g
            iwait(ia, sa)
            istart(k + 1, ib, sb)
            scat(ia)
            iwait(ib, sb)
            istart(k + 2, ia, sa)
            scat(ib)
            return carry

        lax.fori_loop(0, (nb - 1) // 2, pair, 0)
        iwait(ia, sa)
        scat(ia)
        plsc.subcore_barrier()
        pltpu.sync_copy(acc.at[pl.ds(r0, rpt)], out_hbm.at[cid, pl.ds(r0, rpt)])

    f = pl.kernel(
        body,
        out_type=jax.ShapeDtypeStruct((_NUM_CORES, np_, 16), _F32),
        mesh=_mesh(),
        compiler_params=pltpu.CompilerParams(use_tc_tiling_on_sc=False),
        scratch_types=[
            pltpu.VMEM_SHARED((np_, 16), _F32),
            pltpu.VMEM((2, _B), jnp.int32),
            pltpu.VMEM((2, _B), jnp.int32),
            pltpu.VMEM((_B, 16), _F32),
            pltpu.SemaphoreType.DMA,
            pltpu.SemaphoreType.DMA,
        ],
    )
    return f(ep2, ones_b16, zeros16)


def _stream_loop(t_hbm, ep_hbm, acc, ia, ib, b0, b1, sa, sb, sg, jb, nb):
    def istart(k, buf, sem):
        pltpu.async_copy(ep_hbm.at[jb + k], buf, sem)

    def iwait(buf, sem):
        pltpu.make_async_copy(ep_hbm.at[jb], buf, sem).wait()

    def gstart(ibuf, buf):
        pltpu.async_copy(t_hbm.at[ibuf.at[0]], buf, sg)

    def gwait(ibuf, buf):
        pltpu.make_async_copy(t_hbm.at[ibuf.at[0]], buf, sg).wait()

    def scat(ibuf, buf):
        pltpu.sync_copy(buf, acc.at[ibuf.at[1]], add=True)

    istart(0, ia, sa)
    istart(1, ib, sb)
    iwait(ia, sa)
    gstart(ia, b0)

    def pair(g, carry):
        k = 2 * g
        gwait(ia, b0)
        iwait(ib, sb)
        gstart(ib, b1)
        scat(ia, b0)
        istart(k + 2, ia, sa)
        gwait(ib, b1)
        iwait(ia, sa)
        gstart(ia, b0)
        scat(ib, b1)
        istart(k + 3, ib, sb)
        return carry

    n_pairs = (nb - 2) // 2 if nb % 2 == 0 else (nb - 1) // 2
    lax.fori_loop(0, n_pairs, pair, 0)

    if nb % 2 == 0:
        k = nb - 2
        gwait(ia, b0)
        iwait(ib, sb)
        gstart(ib, b1)
        scat(ia, b0)
        gwait(ib, b1)
        scat(ib, b1)
    else:
        gwait(ia, b0)
        scat(ia, b0)
        iwait(ib, sb)


def _sc_spmm_main(g, ep1, zeros_w, n, np_, e, w):
    nb = (e // _B) // _NUM_SUBCORES
    rpt = np_ // _NUM_SUBCORES

    def body(g_hbm, ep_hbm, zeros_hbm, out_hbm,
             acc, ia, ib, b0, b1, sa, sb, sg):
        cid = lax.axis_index("c")
        sid = lax.axis_index("s")
        r0 = sid * rpt
        pltpu.sync_copy(zeros_hbm, acc.at[pl.ds(r0, rpt)])
        plsc.subcore_barrier()
        jb = cid * (e // _B) + sid * nb
        _stream_loop(g_hbm, ep_hbm, acc, ia, ib, b0, b1, sa, sb, sg, jb, nb)
        plsc.subcore_barrier()
        pltpu.sync_copy(acc.at[pl.ds(r0, rpt)], out_hbm.at[cid, pl.ds(r0, rpt)])

    f = pl.kernel(
        body,
        out_type=jax.ShapeDtypeStruct((_NUM_CORES, np_, w), _F32),
        mesh=_mesh(),
        compiler_params=pltpu.CompilerParams(use_tc_tiling_on_sc=False),
        scratch_types=[
            pltpu.VMEM_SHARED((np_, w), _F32),
            pltpu.VMEM((2, _B), jnp.int32),
            pltpu.VMEM((2, _B), jnp.int32),
            pltpu.VMEM((_B, w), _F32),
            pltpu.VMEM((_B, w), _F32),
            pltpu.SemaphoreType.DMA,
            pltpu.SemaphoreType.DMA,
            pltpu.SemaphoreType.DMA,
        ],
    )
    return f(g, ep1, zeros_w)


def _sc_spmm_small(hw2, ep2, zeros16, n, np_, e, c_dim):
    nb = (e // _B) // _NW
    rpt = np_ // _NUM_SUBCORES

    def body(t_hbm, ep_hbm, zeros_hbm, out_hbm,
             acc, ia, ib, b0, b1, sa, sb, sg):
        cid = lax.axis_index("c")
        sid = lax.axis_index("s")
        r0 = sid * rpt
        pltpu.sync_copy(zeros_hbm, acc.at[pl.ds(r0, rpt)])
        plsc.subcore_barrier()
        jb = (cid * _NUM_SUBCORES + sid) * nb
        _stream_loop(t_hbm, ep_hbm, acc, ia, ib, b0, b1, sa, sb, sg, jb, nb)
        plsc.subcore_barrier()
        pltpu.sync_copy(acc.at[pl.ds(r0, rpt)], out_hbm.at[cid, pl.ds(r0, rpt)])

    f = pl.kernel(
        body,
        out_type=jax.ShapeDtypeStruct((_NUM_CORES, np_, c_dim), _F32),
        mesh=_mesh(),
        compiler_params=pltpu.CompilerParams(use_tc_tiling_on_sc=False),
        scratch_types=[
            pltpu.VMEM_SHARED((np_, c_dim), _F32),
            pltpu.VMEM((2, _B), jnp.int32),
            pltpu.VMEM((2, _B), jnp.int32),
            pltpu.VMEM((_B, c_dim), _F32),
            pltpu.VMEM((_B, c_dim), _F32),
            pltpu.SemaphoreType.DMA,
            pltpu.SemaphoreType.DMA,
            pltpu.SemaphoreType.DMA,
        ],
    )
    return f(hw2, ep2, zeros16)


def _tc_tables_body(x_ref, m_ref, deg_ref, g_ref, *, nblk):
    pid = pl.program_id(0)
    deg = deg_ref[0, :, 0:1] + deg_ref[1, :, 0:1]
    dis = jnp.where(deg > 0.0, lax.rsqrt(jnp.maximum(deg, 1e-30)), 0.0)
    xb = x_ref[...]
    x0 = jnp.where(jnp.isnan(xb), 0.0, xb)
    mb = m_ref[...]
    g1m = dis * mb
    g0 = g1m * x0
    bsz = g0.shape[0]
    lane16 = lax.broadcasted_iota(jnp.int32, (bsz, 16), 1)
    pad0 = jnp.zeros((bsz, 16), _F32)
    pad1 = jnp.where(lane16 == 0, jnp.broadcast_to(dis, (bsz, 16)), 0.0)
    blk0 = jnp.concatenate([g0, pad0], axis=1)
    blk1 = jnp.concatenate([g1m, pad1], axis=1)
    g_ref[...] = jnp.where(pid < nblk, blk0, blk1)


def _tc_tables(x, mask, deg16, n, d, w):
    bn = 1000
    nblk = n // bn
    import functools as _ft
    return pl.pallas_call(
        _ft.partial(_tc_tables_body, nblk=nblk),
        grid=(2 * nblk,),
        in_specs=[
            pl.BlockSpec((bn, d), lambda i: (lax.rem(i, nblk), 0)),
            pl.BlockSpec((bn, d), lambda i: (lax.rem(i, nblk), 0)),
            pl.BlockSpec((_NUM_CORES, bn, 16), lambda i: (0, lax.rem(i, nblk), 0)),
        ],
        out_specs=pl.BlockSpec((bn, w), lambda i: (i, 0)),
        out_shape=jax.ShapeDtypeStruct((2 * n, w), _F32),
    )(x, mask, deg16)


def _tc_mlp_body(p_ref, deg_ref, w1_ref, b1_ref, w2_ref, out_ref):
    deg = deg_ref[0, :, 0:1] + deg_ref[1, :, 0:1]
    dis = jnp.where(deg > 0.0, lax.rsqrt(jnp.maximum(deg, 1e-30)), 0.0)
    dis2 = lax.rsqrt(deg + 1.0)
    p0 = p_ref[0, :, 0:128]
    p1m = p_ref[1, :, 0:128]
    s1 = p_ref[1, :, 128:129]
    num = (dis * s1) * p0
    ratio = jnp.where(p1m != 0.0, num / jnp.where(p1m != 0.0, p1m, 1.0), 0.0)
    h = jnp.dot(ratio, w1_ref[...], preferred_element_type=_F32) + b1_ref[...]
    h = jnp.maximum(h, 0.0)
    hw = jnp.dot(h, w2_ref[...], preferred_element_type=_F32)
    out_ref[...] = dis2 * hw


def _tc_mlp(p, deg16, w1, b1, w2, n, np_, w, h_dim, c_dim):
    bn = 1000
    grid = (n // bn,)
    return pl.pallas_call(
        _tc_mlp_body,
        grid=grid,
        in_specs=[
            pl.BlockSpec((_NUM_CORES, bn, w), lambda i: (0, i, 0)),
            pl.BlockSpec((_NUM_CORES, bn, 16), lambda i: (0, i, 0)),
            pl.BlockSpec((128, h_dim), lambda i: (0, 0)),
            pl.BlockSpec((1, h_dim), lambda i: (0, 0)),
            pl.BlockSpec((h_dim, c_dim), lambda i: (0, 0)),
        ],
        out_specs=pl.BlockSpec((bn, c_dim), lambda i: (i, 0)),
        out_shape=jax.ShapeDtypeStruct((n, c_dim), _F32),
    )(p, deg16, w1, b1, w2)


def _tc_final_body(q_ref, hw2_ref, deg_ref, b2_ref, out_ref):
    deg = deg_ref[0, :, 0:1] + deg_ref[1, :, 0:1]
    dis2 = lax.rsqrt(deg + 1.0)
    y = dis2 * (q_ref[0] + q_ref[1] + hw2_ref[...]) + b2_ref[...]
    m = jnp.max(y, axis=1, keepdims=True)
    s = y - m
    out_ref[...] = s - jnp.log(jnp.sum(jnp.exp(s), axis=1, keepdims=True))


def _tc_final(q, hw2, deg16, b2, n, c_dim):
    bn = 1000
    grid = (n // bn,)
    return pl.pallas_call(
        _tc_final_body,
        grid=grid,
        in_specs=[
            pl.BlockSpec((_NUM_CORES, bn, c_dim), lambda i: (0, i, 0)),
            pl.BlockSpec((bn, c_dim), lambda i: (i, 0)),
            pl.BlockSpec((_NUM_CORES, bn, 16), lambda i: (0, i, 0)),
            pl.BlockSpec((1, c_dim), lambda i: (0, 0)),
        ],
        out_specs=pl.BlockSpec((bn, c_dim), lambda i: (i, 0)),
        out_shape=jax.ShapeDtypeStruct((n, c_dim), _F32),
    )(q, hw2, deg16, b2)


def kernel(x, edge_index, mask, W1, b1, W2, b2):
    n, d = x.shape
    e = edge_index.shape[1]
    h_dim = W1.shape[1]
    c_dim = W2.shape[1]
    w = d + 16
    np_ = ((n + _NUM_SUBCORES * 8 - 1) // (_NUM_SUBCORES * 8)) * (_NUM_SUBCORES * 8)
    rpt = np_ // _NUM_SUBCORES

    ep_ = ((e + _NW * _B - 1) // (_NW * _B)) * (_NW * _B)
    npad = ep_ - e
    row = edge_index[0]
    col = edge_index[1]
    nbt = ep_ // _B
    dgat = jnp.arange(npad, dtype=jnp.int32) % n
    dsca = n + (jnp.arange(npad, dtype=jnp.int32) % (np_ - n))
    colg = jnp.reshape(jnp.concatenate([col, dgat]), (nbt, _B))
    rows = jnp.reshape(jnp.concatenate([row, dsca]), (nbt, _B))
    cols = jnp.reshape(jnp.concatenate([col, dsca]), (nbt, _B))
    pad4 = jnp.zeros((4, 2, _B), jnp.int32)
    ep0 = jnp.concatenate([jnp.stack([cols, rows], axis=1), pad4], axis=0)
    ep2 = jnp.concatenate([jnp.stack([colg, rows], axis=1), pad4], axis=0)
    ep1 = jnp.concatenate([
        jnp.stack([colg, rows], axis=1),
        jnp.stack([colg + n, rows], axis=1),
        pad4,
    ], axis=0)

    zeros16 = jnp.zeros((rpt, 16), _F32)
    zeros_w = jnp.zeros((rpt, w), _F32)
    ones_b16 = jnp.ones((_B, 16), _F32)

    deg16 = _sc_degree(ep0, zeros16, ones_b16, n, np_, ep_)
    g = _tc_tables(x, mask, deg16, n, d, w)
    p = _sc_spmm_main(g, ep1, zeros_w, n, np_, ep_, w)
    hw2 = _tc_mlp(p, deg16, W1, jnp.reshape(b1, (1, h_dim)), W2, n, np_, w, h_dim, c_dim)
    q = _sc_spmm_small(hw2, ep2, zeros16, n, np_, ep_, c_dim)
    out = _tc_final(q, hw2, deg16, jnp.reshape(b2, (1, c_dim)), n, c_dim)
    return out


# speedup vs baseline: 1.4290x; 1.0092x over previous
import jax
import jax.numpy as jnp
from jax import lax
from jax.experimental import pallas as pl
from jax.experimental.pallas import tpu as pltpu
from jax.experimental.pallas import tpu_sc as plsc

_F32 = jnp.float32

_NUM_CORES = 2
_NUM_SUBCORES = 16
_NW = _NUM_CORES * _NUM_SUBCORES
_B = 128


def _mesh():
    return plsc.VectorSubcoreMesh(
        core_axis_name="c", subcore_axis_name="s",
        num_cores=_NUM_CORES, num_subcores=_NUM_SUBCORES)


def _vcopy2(srcb, dstb):
    for j in range(2):
        for t in range(_B // 16):
            dstb[j, pl.ds(16 * t, 16)] = srcb[j, pl.ds(16 * t, 16)]


def _sc_degree(ep0, zeros16, ones_b16, n, np_, e):
    nb = (e // _B) // _NW
    rpt = np_ // _NUM_SUBCORES

    def body(ep_hbm, ones_hbm, zeros_hbm, out_hbm,
             acc, ia, ib, pa, pb, ones, sia, sib, ssa, ssb):
        cid = lax.axis_index("c")
        sid = lax.axis_index("s")
        r0 = sid * rpt
        pltpu.sync_copy(zeros_hbm, acc.at[pl.ds(r0, rpt)])
        pltpu.sync_copy(ones_hbm, ones)
        plsc.subcore_barrier()
        jb = (cid * _NUM_SUBCORES + sid) * nb

        def istart(k, ibuf, sem):
            pltpu.async_copy(ep_hbm.at[jb + k], ibuf, sem)

        def iwait(ibuf, sem):
            pltpu.make_async_copy(ep_hbm.at[jb], ibuf, sem).wait()

        def sstart(pbuf, sem):
            pltpu.async_copy(ones, acc.at[pbuf.at[0]], sem, add=True)

        def swait(pbuf, sem):
            pltpu.make_async_copy(ones, acc.at[pbuf.at[0]], sem).wait()

        istart(0, ia, sia)
        istart(1, ib, sib)
        iwait(ia, sia)
        _vcopy2(ia, pa)
        istart(2, ia, sia)
        sstart(pa, ssa)
        iwait(ib, sib)
        _vcopy2(ib, pb)
        istart(3, ib, sib)

        def pair(g, carry):
            k = 2 * g + 1
            sstart(pb, ssb)
            iwait(ia, sia)
            swait(pa, ssa)
            _vcopy2(ia, pa)
            istart(k + 3, ia, sia)
            sstart(pa, ssa)
            iwait(ib, sib)
            swait(pb, ssb)
            _vcopy2(ib, pb)
            istart(k + 4, ib, sib)
            return carry

        lax.fori_loop(0, (nb - 2) // 2, pair, 0)
        sstart(pb, ssb)
        swait(pa, ssa)
        swait(pb, ssb)
        iwait(ia, sia)
        iwait(ib, sib)
        plsc.subcore_barrier()
        pltpu.sync_copy(acc.at[pl.ds(r0, rpt)], out_hbm.at[cid, pl.ds(r0, rpt)])

    f = pl.kernel(
        body,
        out_type=jax.ShapeDtypeStruct((_NUM_CORES, np_, 16), _F32),
        mesh=_mesh(),
        compiler_params=pltpu.CompilerParams(use_tc_tiling_on_sc=False),
        scratch_types=[
            pltpu.VMEM_SHARED((np_, 16), _F32),
            pltpu.VMEM((2, _B), jnp.int32),
            pltpu.VMEM((2, _B), jnp.int32),
            pltpu.VMEM((2, _B), jnp.int32),
            pltpu.VMEM((2, _B), jnp.int32),
            pltpu.VMEM((_B, 16), _F32),
            pltpu.SemaphoreType.DMA,
            pltpu.SemaphoreType.DMA,
            pltpu.SemaphoreType.DMA,
            pltpu.SemaphoreType.DMA,
        ],
    )
    return f(ep0, ones_b16, zeros16)


def _stream_loop(t_hbm, ep_hbm, acc, ia, ib, pa, pb, b0, b1,
                 sia, sib, sg, ssa, ssb, jb, nb):
    def istart(k, ibuf, sem):
        pltpu.async_copy(ep_hbm.at[jb + k], ibuf, sem)

    def iwait(ibuf, sem):
        pltpu.make_async_copy(ep_hbm.at[jb], ibuf, sem).wait()

    def gstart(pbuf, buf):
        pltpu.async_copy(t_hbm.at[pbuf.at[0]], buf, sg)

    def gwait(pbuf, buf):
        pltpu.make_async_copy(t_hbm.at[pbuf.at[0]], buf, sg).wait()

    def sstart(pbuf, buf, sem):
        pltpu.async_copy(buf, acc.at[pbuf.at[1]], sem, add=True)

    def swait(pbuf, buf, sem):
        pltpu.make_async_copy(buf, acc.at[pbuf.at[1]], sem).wait()

    istart(0, ia, sia)
    istart(1, ib, sib)
    iwait(ia, sia)
    _vcopy2(ia, pa)
    istart(2, ia, sia)
    gstart(pa, b0)
    gwait(pa, b0)
    sstart(pa, b0, ssa)
    iwait(ib, sib)
    _vcopy2(ib, pb)
    istart(3, ib, sib)
    gstart(pb, b1)

    def pair(g, carry):
        k = 2 * g + 1
        gwait(pb, b1)
        sstart(pb, b1, ssb)
        iwait(ia, sia)
        swait(pa, b0, ssa)
        _vcopy2(ia, pa)
        istart(k + 3, ia, sia)
        gstart(pa, b0)
        gwait(pa, b0)
        sstart(pa, b0, ssa)
        iwait(ib, sib)
        swait(pb, b1, ssb)
        _vcopy2(ib, pb)
        istart(k + 4, ib, sib)
        gstart(pb, b1)
        return carry

    lax.fori_loop(0, (nb - 2) // 2, pair, 0)
    gwait(pb, b1)
    sstart(pb, b1, ssb)
    swait(pa, b0, ssa)
    swait(pb, b1, ssb)
    iwait(ia, sia)
    iwait(ib, sib)


def _sc_stream_pass(t, ep, zeros, n, np_, w, nb, jb_fn):
    rpt = np_ // _NUM_SUBCORES

    def body(t_hbm, ep_hbm, zeros_hbm, out_hbm,
             acc, ia, ib, pa, pb, b0, b1, sia, sib, sg, ssa, ssb):
        cid = lax.axis_index("c")
        sid = lax.axis_index("s")
        r0 = sid * rpt
        pltpu.sync_copy(zeros_hbm, acc.at[pl.ds(r0, rpt)])
        plsc.subcore_barrier()
        jb = jb_fn(cid, sid)
        _stream_loop(t_hbm, ep_hbm, acc, ia, ib, pa, pb, b0, b1,
                     sia, sib, sg, ssa, ssb, jb, nb)
        plsc.subcore_barrier()
        pltpu.sync_copy(acc.at[pl.ds(r0, rpt)], out_hbm.at[cid, pl.ds(r0, rpt)])

    f = pl.kernel(
        body,
        out_type=jax.ShapeDtypeStruct((_NUM_CORES, np_, w), _F32),
        mesh=_mesh(),
        compiler_params=pltpu.CompilerParams(use_tc_tiling_on_sc=False),
        scratch_types=[
            pltpu.VMEM_SHARED((np_, w), _F32),
            pltpu.VMEM((2, _B), jnp.int32),
            pltpu.VMEM((2, _B), jnp.int32),
            pltpu.VMEM((2, _B), jnp.int32),
            pltpu.VMEM((2, _B), jnp.int32),
            pltpu.VMEM((_B, w), _F32),
            pltpu.VMEM((_B, w), _F32),
            pltpu.SemaphoreType.DMA,
            pltpu.SemaphoreType.DMA,
            pltpu.SemaphoreType.DMA,
            pltpu.SemaphoreType.DMA,
            pltpu.SemaphoreType.DMA,
        ],
    )
    return f(t, ep, zeros)


def _sc_spmm_main(g, ep1, zeros_w, n, np_, e, w):
    nb = (e // _B) // _NUM_SUBCORES
    return _sc_stream_pass(
        g, ep1, zeros_w, n, np_, w, nb,
        lambda cid, sid: cid * (e // _B) + sid * nb)


def _sc_spmm_small(hw2, ep2, zeros16, n, np_, e, c_dim):
    nb = (e // _B) // _NW
    return _sc_stream_pass(
        hw2, ep2, zeros16, n, np_, c_dim, nb,
        lambda cid, sid: (cid * _NUM_SUBCORES + sid) * nb)


def _tc_tables_body(x_ref, m_ref, deg_ref, g_ref, *, nblk):
    pid = pl.program_id(0)
    deg = deg_ref[0, :, 0:1] + deg_ref[1, :, 0:1]
    dis = jnp.where(deg > 0.0, lax.rsqrt(jnp.maximum(deg, 1e-30)), 0.0)
    xb = x_ref[...]
    x0 = jnp.where(jnp.isnan(xb), 0.0, xb)
    mb = m_ref[...]
    g1m = dis * mb
    g0 = g1m * x0
    bsz = g0.shape[0]
    lane16 = lax.broadcasted_iota(jnp.int32, (bsz, 16), 1)
    pad0 = jnp.zeros((bsz, 16), _F32)
    pad1 = jnp.where(lane16 == 0, jnp.broadcast_to(dis, (bsz, 16)), 0.0)
    blk0 = jnp.concatenate([g0, pad0], axis=1)
    blk1 = jnp.concatenate([g1m, pad1], axis=1)
    g_ref[...] = jnp.where(pid < nblk, blk0, blk1)


def _tc_tables(x, mask, deg16, n, d, w):
    bn = 1000
    nblk = n // bn
    import functools as _ft
    return pl.pallas_call(
        _ft.partial(_tc_tables_body, nblk=nblk),
        grid=(2 * nblk,),
        in_specs=[
            pl.BlockSpec((bn, d), lambda i: (lax.rem(i, nblk), 0)),
            pl.BlockSpec((bn, d), lambda i: (lax.rem(i, nblk), 0)),
            pl.BlockSpec((_NUM_CORES, bn, 16), lambda i: (0, lax.rem(i, nblk), 0)),
        ],
        out_specs=pl.BlockSpec((bn, w), lambda i: (i, 0)),
        out_shape=jax.ShapeDtypeStruct((2 * n, w), _F32),
    )(x, mask, deg16)


def _tc_mlp_body(p_ref, deg_ref, w1_ref, b1_ref, w2_ref, out_ref):
    deg = deg_ref[0, :, 0:1] + deg_ref[1, :, 0:1]
    dis = jnp.where(deg > 0.0, lax.rsqrt(jnp.maximum(deg, 1e-30)), 0.0)
    dis2 = lax.rsqrt(deg + 1.0)
    p0 = p_ref[0, :, 0:128]
    p1m = p_ref[1, :, 0:128]
    s1 = p_ref[1, :, 128:129]
    num = (dis * s1) * p0
    ratio = jnp.where(p1m != 0.0, num / jnp.where(p1m != 0.0, p1m, 1.0), 0.0)
    h = jnp.dot(ratio, w1_ref[...], preferred_element_type=_F32) + b1_ref[...]
    h = jnp.maximum(h, 0.0)
    hw = jnp.dot(h, w2_ref[...], preferred_element_type=_F32)
    out_ref[...] = dis2 * hw


def _tc_mlp(p, deg16, w1, b1, w2, n, np_, w, h_dim, c_dim):
    bn = 1000
    grid = (n // bn,)
    return pl.pallas_call(
        _tc_mlp_body,
        grid=grid,
        in_specs=[
            pl.BlockSpec((_NUM_CORES, bn, w), lambda i: (0, i, 0)),
            pl.BlockSpec((_NUM_CORES, bn, 16), lambda i: (0, i, 0)),
            pl.BlockSpec((128, h_dim), lambda i: (0, 0)),
            pl.BlockSpec((1, h_dim), lambda i: (0, 0)),
            pl.BlockSpec((h_dim, c_dim), lambda i: (0, 0)),
        ],
        out_specs=pl.BlockSpec((bn, c_dim), lambda i: (i, 0)),
        out_shape=jax.ShapeDtypeStruct((n, c_dim), _F32),
    )(p, deg16, w1, b1, w2)


def _tc_final_body(q_ref, hw2_ref, deg_ref, b2_ref, out_ref):
    deg = deg_ref[0, :, 0:1] + deg_ref[1, :, 0:1]
    dis2 = lax.rsqrt(deg + 1.0)
    y = dis2 * (q_ref[0] + q_ref[1] + hw2_ref[...]) + b2_ref[...]
    m = jnp.max(y, axis=1, keepdims=True)
    s = y - m
    out_ref[...] = s - jnp.log(jnp.sum(jnp.exp(s), axis=1, keepdims=True))


def _tc_final(q, hw2, deg16, b2, n, c_dim):
    bn = 1000
    grid = (n // bn,)
    return pl.pallas_call(
        _tc_final_body,
        grid=grid,
        in_specs=[
            pl.BlockSpec((_NUM_CORES, bn, c_dim), lambda i: (0, i, 0)),
            pl.BlockSpec((bn, c_dim), lambda i: (i, 0)),
            pl.BlockSpec((_NUM_CORES, bn, 16), lambda i: (0, i, 0)),
            pl.BlockSpec((1, c_dim), lambda i: (0, 0)),
        ],
        out_specs=pl.BlockSpec((bn, c_dim), lambda i: (i, 0)),
        out_shape=jax.ShapeDtypeStruct((n, c_dim), _F32),
    )(q, hw2, deg16, b2)


def kernel(x, edge_index, mask, W1, b1, W2, b2):
    n, d = x.shape
    e = edge_index.shape[1]
    h_dim = W1.shape[1]
    c_dim = W2.shape[1]
    w = d + 16
    np_ = ((n + _NUM_SUBCORES * 8 - 1) // (_NUM_SUBCORES * 8)) * (_NUM_SUBCORES * 8)
    rpt = np_ // _NUM_SUBCORES

    ep_ = ((e + _NW * 80 * _B - 1) // (_NW * 80 * _B)) * (_NW * 80 * _B)
    npad = ep_ - e
    row = edge_index[0]
    col = edge_index[1]
    nbt = ep_ // _B
    dgat = jnp.arange(npad, dtype=jnp.int32) % n
    dsca = n + (jnp.arange(npad, dtype=jnp.int32) % (np_ - n))
    colg = jnp.reshape(jnp.concatenate([col, dgat]), (nbt, _B))
    rows = jnp.reshape(jnp.concatenate([row, dsca]), (nbt, _B))
    cols = jnp.reshape(jnp.concatenate([col, dsca]), (nbt, _B))
    pad4 = jnp.zeros((4, 2, _B), jnp.int32)
    ep0 = jnp.concatenate([jnp.stack([cols, rows], axis=1), pad4], axis=0)
    ep2 = jnp.concatenate([jnp.stack([colg, rows], axis=1), pad4], axis=0)
    ep1 = jnp.concatenate([
        jnp.stack([colg, rows], axis=1),
        jnp.stack([colg + n, rows], axis=1),
        pad4,
    ], axis=0)

    zeros16 = jnp.zeros((rpt, 16), _F32)
    zeros_w = jnp.zeros((rpt, w), _F32)
    ones_b16 = jnp.ones((_B, 16), _F32)

    deg16 = _sc_degree(ep0, zeros16, ones_b16, n, np_, ep_)
    g = _tc_tables(x, mask, deg16, n, d, w)
    p = _sc_spmm_main(g, ep1, zeros_w, n, np_, ep_, w)
    hw2 = _tc_mlp(p, deg16, W1, jnp.reshape(b1, (1, h_dim)), W2, n, np_, w, h_dim, c_dim)
    q = _sc_spmm_small(hw2, ep2, zeros16, n, np_, ep_, c_dim)
    out = _tc_final(q, hw2, deg16, jnp.reshape(b2, (1, c_dim)), n, c_dim)
    return out
